# Initial kernel scaffold; baseline (speedup 1.0000x reference)
import jax
import jax.numpy as jnp
from jax.experimental import pallas as pl


def kernel(x, adj_norm, W_in, W_h, W_mean, W_logstd, g0, b0, g1, b1, gm, bm, gl, bl, eps):
    raise NotImplementedError("write your pallas kernel here")


# speedup vs baseline: 1.2698x; 1.2698x over previous
import functools

import jax
import jax.numpy as jnp
from jax.experimental import pallas as pl
from jax.experimental.pallas import tpu as pltpu


def _pick_block(n, candidates):
    for c in candidates:
        if n % c == 0:
            return c
    return n


def _ln(t, g, b):
    m = jnp.mean(t, axis=-1, keepdims=True)
    v = jnp.mean((t - m) * (t - m), axis=-1, keepdims=True)
    return (t - m) * jax.lax.rsqrt(v + 1e-5) * g + b


def _p0_body(x_ref, win_ref, p0_ref):
    p0_ref[...] = jnp.dot(x_ref[...], win_ref[...],
                          preferred_element_type=jnp.float32)


def _p1_body(a_ref, p0_ref, wh_ref, g0_ref, b0_ref, h0_ref, p1_ref):
    t = jnp.dot(a_ref[...], p0_ref[...], preferred_element_type=jnp.float32)
    h = _ln(jnp.maximum(t, 0.0), g0_ref[...], b0_ref[...])
    h0_ref[...] = h
    p1_ref[...] = jnp.dot(h, wh_ref[...], preferred_element_type=jnp.float32)


def _p2_body(a_ref, p1_ref, h0_ref, wml_ref, g1_ref, b1_ref, p2_ref):
    t = jnp.dot(a_ref[...], p1_ref[...], preferred_element_type=jnp.float32)
    h1 = _ln(jnp.maximum(t, 0.0), g1_ref[...], b1_ref[...]) + h0_ref[...]
    p2_ref[...] = jnp.dot(h1, wml_ref[...], preferred_element_type=jnp.float32)


def _p3_body(a_ref, p2_ref, eps_ref, gm_ref, bm_ref, gl_ref, bl_ref, z_ref, *, h2):
    t = jnp.dot(a_ref[...], p2_ref[...], preferred_element_type=jnp.float32)
    mean = _ln(t[:, :h2], gm_ref[...], bm_ref[...])
    logstd = _ln(t[:, h2:], gl_ref[...], bl_ref[...])
    z_ref[...] = eps_ref[...] * jnp.exp(logstd) + mean


def _p4_body(zr_ref, zc_ref, out_ref):
    s = jax.lax.dot_general(
        zr_ref[...], zc_ref[...],
        dimension_numbers=(((1,), (1,)), ((), ())),
        preferred_element_type=jnp.float32,
    )
    out_ref[...] = jax.nn.sigmoid(s)


def kernel(x, adj_norm, W_in, W_h, W_mean, W_logstd,
           g0, b0, g1, b1, gm, bm, gl, bl, eps):
    n, d = x.shape
    h1d = W_in.shape[1]
    h2 = W_mean.shape[1]
    br = _pick_block(n, (400, 200, 100, 8))
    nb = n // br

    g0r, b0r = g0.reshape(1, -1), b0.reshape(1, -1)
    g1r, b1r = g1.reshape(1, -1), b1.reshape(1, -1)
    gmr, bmr = gm.reshape(1, -1), bm.reshape(1, -1)
    glr, blr = gl.reshape(1, -1), bl.reshape(1, -1)
    wml = jnp.concatenate([W_mean, W_logstd], axis=1)

    row_spec = pl.BlockSpec((br, n), lambda i: (i, 0))
    full = lambda shape: pl.BlockSpec(shape, lambda i: tuple(0 for _ in shape))
    out_row = lambda w: pl.BlockSpec((br, w), lambda i: (i, 0))

    p0 = pl.pallas_call(
        _p0_body,
        in_specs=[pl.BlockSpec((n, d), lambda: (0, 0)),
                  pl.BlockSpec((d, h1d), lambda: (0, 0))],
        out_specs=pl.BlockSpec((n, h1d), lambda: (0, 0)),
        out_shape=jax.ShapeDtypeStruct((n, h1d), jnp.float32),
    )(x, W_in)

    h0, p1 = pl.pallas_call(
        _p1_body,
        grid=(nb,),
        in_specs=[row_spec, full((n, h1d)), full((h1d, h1d)),
                  full((1, h1d)), full((1, h1d))],
        out_specs=[out_row(h1d), out_row(h1d)],
        out_shape=[jax.ShapeDtypeStruct((n, h1d), jnp.float32),
                   jax.ShapeDtypeStruct((n, h1d), jnp.float32)],
    )(adj_norm, p0, W_h, g0r, b0r)

    p2 = pl.pallas_call(
        _p2_body,
        grid=(nb,),
        in_specs=[row_spec, full((n, h1d)), out_row(h1d), full((h1d, 2 * h2)),
                  full((1, h1d)), full((1, h1d))],
        out_specs=out_row(2 * h2),
        out_shape=jax.ShapeDtypeStruct((n, 2 * h2), jnp.float32),
    )(adj_norm, p1, h0, wml, g1r, b1r)

    z = pl.pallas_call(
        functools.partial(_p3_body, h2=h2),
        grid=(nb,),
        in_specs=[row_spec, full((n, 2 * h2)), out_row(h2),
                  full((1, h2)), full((1, h2)), full((1, h2)), full((1, h2))],
        out_specs=out_row(h2),
        out_shape=jax.ShapeDtypeStruct((n, h2), jnp.float32),
    )(adj_norm, p2, eps, gmr, bmr, glr, blr)

    bm_ = _pick_block(n, (400, 200, 100, 8))
    out = pl.pallas_call(
        _p4_body,
        grid=(n // bm_,),
        in_specs=[pl.BlockSpec((bm_, h2), lambda i: (i, 0)),
                  pl.BlockSpec((n, h2), lambda i: (0, 0))],
        out_specs=pl.BlockSpec((bm_, n), lambda i: (i, 0)),
        out_shape=jax.ShapeDtypeStruct((n, n), jnp.float32),
    )(z, z)
    return out


# speedup vs baseline: 1.2705x; 1.0006x over previous
import functools

import jax
import jax.numpy as jnp
from jax.experimental import pallas as pl
from jax.experimental.pallas import tpu as pltpu


def _pick_block(n, candidates):
    for c in candidates:
        if n % c == 0:
            return c
    return n


def _ln(t, g, b):
    m = jnp.mean(t, axis=-1, keepdims=True)
    v = jnp.mean((t - m) * (t - m), axis=-1, keepdims=True)
    return (t - m) * jax.lax.rsqrt(v + 1e-5) * g + b


def _p0_body(x_ref, win_ref, p0_ref):
    p0_ref[...] = jnp.dot(x_ref[...], win_ref[...],
                          preferred_element_type=jnp.float32)


def _p1_body(a_ref, p0_ref, wh_ref, g0_ref, b0_ref, h0_ref, p1_ref):
    t = jnp.dot(a_ref[...], p0_ref[...], preferred_element_type=jnp.float32)
    h = _ln(jnp.maximum(t, 0.0), g0_ref[...], b0_ref[...])
    h0_ref[...] = h
    p1_ref[...] = jnp.dot(h, wh_ref[...], preferred_element_type=jnp.float32)


def _p2_body(a_ref, p1_ref, h0_ref, wml_ref, g1_ref, b1_ref, p2_ref):
    t = jnp.dot(a_ref[...], p1_ref[...], preferred_element_type=jnp.float32)
    h1 = _ln(jnp.maximum(t, 0.0), g1_ref[...], b1_ref[...]) + h0_ref[...]
    p2_ref[...] = jnp.dot(h1, wml_ref[...], preferred_element_type=jnp.float32)


def _p3_body(a_ref, p2_ref, eps_ref, gm_ref, bm_ref, gl_ref, bl_ref, z_ref,
             *, h2):
    t = jnp.dot(a_ref[...], p2_ref[...], preferred_element_type=jnp.float32)
    mean = _ln(t[:, :h2], gm_ref[...], bm_ref[...])
    logstd = _ln(t[:, h2:], gl_ref[...], bl_ref[...])
    z_ref[...] = eps_ref[...] * jnp.exp(logstd) + mean


def _p4_body(zr_ref, zc_ref, out_ref):
    s = jax.lax.dot_general(
        zr_ref[...], zc_ref[...],
        dimension_numbers=(((1,), (1,)), ((), ())),
        preferred_element_type=jnp.float32,
    )
    out_ref[...] = jax.nn.sigmoid(s)


def kernel(x, adj_norm, W_in, W_h, W_mean, W_logstd,
           g0, b0, g1, b1, gm, bm, gl, bl, eps):
    n, d = x.shape
    h1d = W_in.shape[1]
    h2 = W_mean.shape[1]
    br = _pick_block(n, (400, 200, 100, 8))
    nb = n // br

    g0r, b0r = g0.reshape(1, -1), b0.reshape(1, -1)
    g1r, b1r = g1.reshape(1, -1), b1.reshape(1, -1)
    gmr, bmr = gm.reshape(1, -1), bm.reshape(1, -1)
    glr, blr = gl.reshape(1, -1), bl.reshape(1, -1)
    wml = jnp.concatenate([W_mean, W_logstd], axis=1)

    row_spec = pl.BlockSpec((br, n), lambda i: (i, 0))
    full = lambda shape: pl.BlockSpec(shape, lambda i: tuple(0 for _ in shape))
    out_row = lambda w: pl.BlockSpec((br, w), lambda i: (i, 0))

    p0 = pl.pallas_call(
        _p0_body,
        in_specs=[pl.BlockSpec((n, d), lambda: (0, 0)),
                  pl.BlockSpec((d, h1d), lambda: (0, 0))],
        out_specs=pl.BlockSpec((n, h1d), lambda: (0, 0)),
        out_shape=jax.ShapeDtypeStruct((n, h1d), jnp.float32),
    )(x, W_in)

    h0, p1 = pl.pallas_call(
        _p1_body,
        grid=(nb,),
        in_specs=[row_spec, full((n, h1d)), full((h1d, h1d)),
                  full((1, h1d)), full((1, h1d))],
        out_specs=[out_row(h1d), out_row(h1d)],
        out_shape=[jax.ShapeDtypeStruct((n, h1d), jnp.float32),
                   jax.ShapeDtypeStruct((n, h1d), jnp.float32)],
    )(adj_norm, p0, W_h, g0r, b0r)

    p2 = pl.pallas_call(
        _p2_body,
        grid=(nb,),
        in_specs=[row_spec, full((n, h1d)), out_row(h1d), full((h1d, 2 * h2)),
                  full((1, h1d)), full((1, h1d))],
        out_specs=out_row(2 * h2),
        out_shape=jax.ShapeDtypeStruct((n, 2 * h2), jnp.float32),
    )(adj_norm, p1, h0, wml, g1r, b1r)

    z = pl.pallas_call(
        functools.partial(_p3_body, h2=h2),
        grid=(nb,),
        in_specs=[row_spec, full((n, 2 * h2)), out_row(h2),
                  full((1, h2)), full((1, h2)), full((1, h2)), full((1, h2))],
        out_specs=out_row(h2),
        out_shape=jax.ShapeDtypeStruct((n, h2), jnp.float32),
    )(adj_norm, p2, eps, gmr, bmr, glr, blr)

    out = pl.pallas_call(
        _p4_body,
        grid=(nb,),
        in_specs=[out_row(h2), pl.BlockSpec((n, h2), lambda i: (0, 0))],
        out_specs=row_spec,
        out_shape=jax.ShapeDtypeStruct((n, n), jnp.float32),
    )(z, z)
    return out
